# baseline (speedup 1.0000x reference)
import functools

import jax
import jax.numpy as jnp
from jax import lax
from jax.experimental import pallas as pl
from jax.experimental.pallas import tpu as pltpu
from jax.experimental.pallas import tpu_sc as plsc

DIM = 64
PAD = 128
NSPECIAL = 101
RB = 8192
SB = 999904


def _pack_kernel(f2_ref, f1_ref, unk_ref, o_ref):
    g = pl.program_id(0)
    o_ref[:, :DIM] = f2_ref[...]

    @pl.when(g == pl.num_programs(0) - 1)
    def _():
        base = SB - (pl.num_programs(0) - 1) * RB
        o_ref[pl.ds(base, 100), :DIM] = f1_ref[...]
        o_ref[pl.ds(base + 100, 1), :DIM] = unk_ref[...]


def kernel(input, frozen1, unk, frozen2):
    B, L = input.shape
    N = B * L
    NF2 = frozen2.shape[0]
    n_blk = (SB + RB) // RB
    TT = n_blk * RB

    tbl = pl.pallas_call(
        _pack_kernel,
        grid=(n_blk,),
        in_specs=[
            pl.BlockSpec((RB, DIM), lambda g: (g, 0)),
            pl.BlockSpec((100, DIM), lambda g: (0, 0)),
            pl.BlockSpec((1, DIM), lambda g: (0, 0)),
        ],
        out_specs=pl.BlockSpec((RB, PAD), lambda g: (g, 0)),
        out_shape=jax.ShapeDtypeStruct((TT, PAD), jnp.float32),
        compiler_params=pltpu.CompilerParams(
            dimension_semantics=("parallel",)),
    )(frozen2, frozen1, unk)

    info = plsc.get_sparse_core_info()
    NC, NS = info.num_cores, info.num_subcores
    NW = NC * NS
    b_per_w = B // NW
    BCH = 4
    n_ch = b_per_w // BCH

    mesh = plsc.VectorSubcoreMesh(core_axis_name="c", subcore_axis_name="s")

    @functools.partial(
        pl.kernel,
        mesh=mesh,
        out_type=jax.ShapeDtypeStruct((N, PAD), jnp.float32),
        scratch_types=[
            pltpu.VMEM((b_per_w, L), jnp.int32),
            pltpu.VMEM((b_per_w, L), jnp.int32),
            pltpu.VMEM((BCH * L, PAD), jnp.float32),
            pltpu.VMEM((BCH * L, PAD), jnp.float32),
            pltpu.SemaphoreType.DMA,
            pltpu.SemaphoreType.DMA,
        ],
        compiler_params=pltpu.CompilerParams(
            use_tc_tiling_on_sc=True, needs_layout_passes=False),
    )
    def kern(idx_hbm, tbl_hbm, out_hbm, idx_v, gidx_v, rows0, rows1,
             sem_g, sem_o):
        wid = lax.axis_index("s") * NC + lax.axis_index("c")
        wb = wid * b_per_w
        wr = wb * L

        pltpu.sync_copy(idx_hbm.at[pl.ds(wb, b_per_w)], idx_v)

        def remap(b, carry):
            for off in (0, 16, 32, L - 16):
                v = idx_v[b, pl.ds(off, 16)]
                gidx_v[b, pl.ds(off, 16)] = jnp.where(
                    v >= NSPECIAL, v - NSPECIAL, v + SB)
            return carry
        lax.fori_loop(0, b_per_w, remap, 0)

        rows = (rows0, rows1)

        def p2_body(st, carry):
            for rb in range(2):
                ch = st * 2 + rb

                @pl.when(ch >= 2)
                def _():
                    pltpu.make_async_copy(
                        rows[rb], out_hbm.at[pl.ds(wr, BCH * L)], sem_o).wait()
                handles = []
                for bb in range(BCH):
                    b = ch * BCH + bb
                    handles.append(pltpu.async_copy(
                        tbl_hbm.at[gidx_v.at[b, pl.ds(0, L)]],
                        rows[rb].at[pl.ds(bb * L, L)], sem_g))
                for h in handles:
                    h.wait()
                pltpu.async_copy(
                    rows[rb], out_hbm.at[pl.ds(wr + ch * BCH * L, BCH * L)],
                    sem_o)
            return carry
        lax.fori_loop(0, n_ch // 2, p2_body, 0)
        for _ in range(2):
            pltpu.make_async_copy(
                rows[0], out_hbm.at[pl.ds(wr, BCH * L)], sem_o).wait()

    packed = kern(input, tbl)
    return packed[:, :DIM].reshape(B, L, DIM)


# speedup vs baseline: 1.2030x; 1.2030x over previous
import functools

import jax
import jax.numpy as jnp
from jax import lax
from jax.experimental import pallas as pl
from jax.experimental.pallas import tpu as pltpu
from jax.experimental.pallas import tpu_sc as plsc

DIM = 64
NSPECIAL = 101
LANES = 16


def kernel(input, frozen1, unk, frozen2):
    B, L = input.shape
    N = B * L
    info = plsc.get_sparse_core_info()
    NC, NS = info.num_cores, info.num_subcores
    NW = NC * NS
    n_per_w = N // NW
    SUB = 128
    FIRE = 5
    CHUNK = SUB * FIRE
    n_chunks = n_per_w // CHUNK
    n_groups = n_per_w // LANES
    gpc = CHUNK // LANES

    idx_flat = input.reshape(N)
    mesh = plsc.VectorSubcoreMesh(core_axis_name="c", subcore_axis_name="s")

    @functools.partial(
        pl.kernel,
        mesh=mesh,
        out_type=jax.ShapeDtypeStruct((N, DIM), jnp.float32),
        scratch_types=[
            pltpu.VMEM((n_per_w,), jnp.int32),
            pltpu.VMEM((n_per_w,), jnp.int32),
            pltpu.VMEM((NSPECIAL + 3, DIM), jnp.float32),
            pltpu.VMEM((CHUNK, DIM), jnp.float32),
            pltpu.VMEM((CHUNK, DIM), jnp.float32),
            pltpu.SMEM((n_groups,), jnp.int32),
            pltpu.SemaphoreType.DMA,
            pltpu.SemaphoreType.DMA,
        ],
        compiler_params=pltpu.CompilerParams(
            use_tc_tiling_on_sc=False, needs_layout_passes=False),
    )
    def kern(idx_hbm, f1_hbm, unk_hbm, f2_hbm, out_hbm,
             idx_v, gidx_v, small_v, rows0, rows1, cnt_s, sem_g, sem_o):
        wid = lax.axis_index("s") * NC + lax.axis_index("c")
        base = wid * n_per_w

        pltpu.sync_copy(f1_hbm, small_v.at[pl.ds(0, 100)])
        pltpu.sync_copy(unk_hbm, small_v.at[pl.ds(100, 1)])
        pltpu.sync_copy(idx_hbm.at[pl.ds(base, n_per_w)], idx_v)

        def prep(g, carry):
            v = idx_v[pl.ds(g * LANES, LANES)]
            sp = v < NSPECIAL
            gidx_v[pl.ds(g * LANES, LANES)] = jnp.where(sp, 0, v - NSPECIAL)
            cnt_s[g] = jnp.sum(jnp.where(sp, 1, 0))
            return carry
        lax.fori_loop(0, n_groups, prep, 0)

        rows = (rows0, rows1)

        def chunk_body(st, carry):
            for rb in range(2):
                c = st * 2 + rb
                off = c * CHUNK

                @pl.when(c >= 2)
                def _():
                    pltpu.make_async_copy(
                        rows[rb], out_hbm.at[pl.ds(base, CHUNK)], sem_o).wait()
                handles = []
                for s in range(FIRE):
                    handles.append(pltpu.async_copy(
                        f2_hbm.at[gidx_v.at[pl.ds(off + s * SUB, SUB)]],
                        rows[rb].at[pl.ds(s * SUB, SUB)],
                        sem_g))
                for h in handles:
                    h.wait()

                def fix_group(g, gcarry, _rows=rows[rb], _c=c):
                    gg = _c * gpc + g

                    @pl.when(cnt_s[gg] > 0)
                    def _():
                        v = idx_v[pl.ds(gg * LANES, LANES)]
                        m = v < NSPECIAL
                        sidx = jnp.where(m, v, 0)
                        rowpos = g * LANES + lax.iota(jnp.int32, LANES)

                        def fix_col(col, ccarry):
                            cvec = jnp.full((LANES,), col, jnp.int32)
                            vals = plsc.load_gather(small_v, [sidx, cvec], mask=m)
                            plsc.store_scatter(_rows, [rowpos, cvec], vals, mask=m)
                            return ccarry
                        lax.fori_loop(0, DIM, fix_col, 0)
                    return gcarry
                lax.fori_loop(0, gpc, fix_group, 0)

                pltpu.async_copy(rows[rb], out_hbm.at[pl.ds(base + off, CHUNK)],
                                 sem_o)
            return carry
        lax.fori_loop(0, n_chunks // 2, chunk_body, 0)
        for _ in range(2):
            pltpu.make_async_copy(
                rows[0], out_hbm.at[pl.ds(base, CHUNK)], sem_o).wait()

    out = kern(idx_flat, frozen1, unk, frozen2)
    return out.reshape(B, L, DIM)
